# baseline (speedup 1.0000x reference)
import functools

import jax
import jax.numpy as jnp
from jax import lax
from jax.experimental import pallas as pl
from jax.experimental.pallas import tpu as pltpu
from jax.experimental.pallas import tpu_sc as plsc

K = 32
QB = 256
C_TAB = 128
MIN32 = -2147483648


def _dist_body(q_ref, db_ref, keys_ref, thr_ref):
    q = q_ref[0]
    db = db_ref[0]
    s1 = jnp.sum(q * q, axis=0, keepdims=True)
    s2 = jnp.sum(db * db, axis=1, keepdims=True)
    mm = lax.dot_general(db.astype(jnp.bfloat16), q.astype(jnp.bfloat16),
                         (((1,), (0,)), ((), ())),
                         preferred_element_type=jnp.float32)
    d = (s1 + s2) - 2.0 * mm
    x = lax.bitcast_convert_type(d, jnp.int32)
    skey = x ^ (lax.shift_right_arithmetic(x, 31) & 0x7FFFFFFF)

    def body(i, prefix):
        bitval = lax.shift_left(jnp.int32(1), 31 - i)
        cand = prefix | bitval
        cand_s = cand ^ MIN32
        cnt = jnp.sum((skey < cand_s).astype(jnp.int32), axis=0, keepdims=True)
        return jnp.where(cnt >= K, prefix, cand)

    prefix = lax.fori_loop(0, 32, body, jnp.zeros((1, QB), jnp.int32))
    keys_ref[0] = skey
    thr_ref[0] = prefix ^ MIN32


def _dist_thresh(q3, db, blocks_per_batch):
    G = q3.shape[0]
    N = db.shape[1]
    keys, thr = pl.pallas_call(
        _dist_body,
        grid=(G,),
        in_specs=[pl.BlockSpec((1, 3, QB), lambda g: (g, 0, 0)),
                  pl.BlockSpec((1, N, 3),
                               lambda g, _n=blocks_per_batch: (g // _n, 0, 0))],
        out_specs=[pl.BlockSpec((1, N, QB), lambda g: (g, 0, 0)),
                   pl.BlockSpec((1, 1, QB), lambda g: (g, 0, 0))],
        out_shape=[jax.ShapeDtypeStruct((G, N, QB), jnp.int32),
                   jax.ShapeDtypeStruct((G, 1, QB), jnp.int32)],
    )(q3, db)
    return keys, thr


CCH = 64
SBQ = 4


def _sc_body(keys_hbm, thr_hbm, table_hbm, out_hbm,
             keys_v, thr_v, bufA, bufB, cntA_v, cntB_v, idx_v, rows_v,
             semg, semw, *, n_cand, q_total, s_batch):
    nc = 2
    wid = lax.axis_index("s") * nc + lax.axis_index("c")
    blk = wid
    base = blk * QB
    boff = (base // s_batch) * n_cand
    lane = lax.broadcasted_iota(jnp.int32, (16,), 0)
    zeros16 = jnp.zeros((16,), jnp.int32)

    pltpu.sync_copy(thr_hbm.at[pl.ds(blk, 1), :], thr_v)

    def zrow(i, _):
        cntA_v[i] = zeros16
        cntB_v[i] = zeros16
        return 0

    lax.fori_loop(0, 16, zrow, 0)

    def chunk_body(ch, _):
        pltpu.sync_copy(keys_hbm.at[pl.ds(blk, 1), pl.ds(ch * CCH, CCH), :],
                        keys_v)

        def qg_body(qg, _):
            t = thr_v[0, pl.ds(qg * 16, 16)]
            qabs = qg * 16 + lane
            a49 = qabs * 49
            a65 = qabs * 65

            def cbody(c, carry):
                cA, cB = carry
                k = keys_v[0, c, pl.ds(qg * 16, 16)]
                mlt = k < t
                meq = k == t
                pos = lax.broadcast(c + ch * CCH + boff, (16,))
                plsc.store_scatter(bufA, [a49 + cA], pos, mask=mlt)
                plsc.store_scatter(bufB, [a65 + cB], pos, mask=meq)
                return (cA + mlt.astype(jnp.int32),
                        jnp.minimum(cB + meq.astype(jnp.int32), 64))

            cA, cB = plsc.parallel_loop(
                0, CCH, unroll=8,
                carry=(cntA_v[qg], cntB_v[qg]))(cbody)
            cntA_v[qg] = cA
            cntB_v[qg] = cB
            return 0

        lax.fori_loop(0, 16, qg_body, 0)
        return 0

    lax.fori_loop(0, n_cand // CCH, chunk_body, 0)

    def mq(q, _):
        ca_vec = cntA_v[q // 16]
        ca = ca_vec[lax.broadcast(q % 16, (16,))]
        for half in range(2):
            j = lane + 16 * half
            vA = plsc.load_gather(bufA, [q * 49 + j])
            jb = jnp.maximum(j - ca, 0)
            vB = plsc.load_gather(bufB, [q * 65 + jb])
            sel = jnp.where(j < ca, vA, vB)
            idx_v[q, pl.ds(16 * half, 16)] = sel
        return 0

    lax.fori_loop(0, QB, mq, 0)

    nsb = QB // SBQ
    gbytes_dst = lambda s, r: rows_v.at[s, pl.ds(r * K, K)]

    def fire_g(i, s):
        for r in range(SBQ):
            pltpu.async_copy(table_hbm.at[idx_v.at[i * SBQ + r]],
                             gbytes_dst(s, r), semg[s])

    def drain_g(s):
        for r in range(SBQ):
            pltpu.make_async_copy(table_hbm.at[pl.ds(0, K)],
                                  gbytes_dst(s, r), semg[s]).wait()

    def fire_w(i, s):
        pltpu.async_copy(rows_v.at[s],
                         out_hbm.at[pl.ds((base + i * SBQ) * K, SBQ * K)],
                         semw[s])

    def drain_w(s):
        pltpu.make_async_copy(rows_v.at[s],
                              out_hbm.at[pl.ds(base * K, SBQ * K)],
                              semw[s]).wait()

    fire_g(0, 0)

    def pipe(g, _):
        @pl.when(g > 0)
        def _():
            drain_w(1)
        fire_g(2 * g + 1, 1)
        drain_g(0)
        fire_w(2 * g, 0)

        @pl.when(g < nsb // 2 - 1)
        def _():
            drain_w(0)
            fire_g(2 * g + 2, 0)

        drain_g(1)
        fire_w(2 * g + 1, 1)
        return 0

    lax.fori_loop(0, nsb // 2, pipe, 0)
    drain_w(0)
    drain_w(1)


def _sc_extract_gather(keys, thr, table, q_total, s_batch):
    n_cand = keys.shape[1]
    body = functools.partial(_sc_body, n_cand=n_cand, q_total=q_total,
                             s_batch=s_batch)
    fn = functools.partial(
        pl.kernel, body,
        mesh=plsc.VectorSubcoreMesh(core_axis_name="c", subcore_axis_name="s"),
        compiler_params=pltpu.CompilerParams(needs_layout_passes=False),
        out_type=jax.ShapeDtypeStruct((q_total * K, C_TAB), jnp.float32),
        scratch_types=[
            pltpu.VMEM((1, CCH, QB), jnp.int32),
            pltpu.VMEM((1, QB), jnp.int32),
            pltpu.VMEM((QB * 49,), jnp.int32),
            pltpu.VMEM((QB * 65,), jnp.int32),
            pltpu.VMEM((16, 16), jnp.int32),
            pltpu.VMEM((16, 16), jnp.int32),
            pltpu.VMEM((QB, K), jnp.int32),
            pltpu.VMEM((2, SBQ * K, C_TAB), jnp.float32),
            [pltpu.SemaphoreType.DMA, pltpu.SemaphoreType.DMA],
            [pltpu.SemaphoreType.DMA, pltpu.SemaphoreType.DMA],
        ])()
    return fn(keys, thr, table)


def _dot(a, b):
    return lax.dot_general(a.astype(jnp.bfloat16), b.astype(jnp.bfloat16),
                           (((1,), (0,)), ((), ())),
                           preferred_element_type=jnp.float32)


def _pair_geom(g_ref, p_ref, bs):
    G = g_ref[...]
    p = p_ref[...]
    p3 = jnp.broadcast_to(p[:, None, :], (bs, K, 3)).reshape(bs * K, 3)
    qx = G[:, 64:67]
    diff = qx - p3
    euc = jnp.sqrt(jnp.sum(diff * diff, axis=1, keepdims=True) + 1e-20)
    xyz10 = jnp.concatenate([p3, qx, diff, euc], axis=1)
    return G[:, 0:64], xyz10


def _softmax_sum(A, F, bs):
    A3 = A.reshape(bs, K, 64)
    F3 = F.reshape(bs, K, 64)
    m = jnp.max(A3, axis=1, keepdims=True)
    e = jnp.exp(A3 - m)
    w = e / jnp.sum(e, axis=1, keepdims=True)
    return jnp.sum(w * F3, axis=1)


def _phase1_body(g_ref, p_ref, f1_ref, w1a_x, w1a_f1, w1a_f2, b1a, w1b, b1b,
                 w1c, b1c, wx1, bx1, w2a_e, w2a_f, b2a, w2b, b2b, o_ref, *, bs):
    qf, xyz10 = _pair_geom(g_ref, p_ref, bs)
    f1 = f1_ref[...]
    f1r = jnp.broadcast_to(f1[:, None, :], (bs, K, 64)).reshape(bs * K, 64)
    h = jnp.maximum(_dot(xyz10, w1a_x[...]) + _dot(f1r, w1a_f1[...])
                    + _dot(qf, w1a_f2[...]) + b1a[...], 0.0)
    h = jnp.maximum(_dot(h, w1b[...]) + b1b[...], 0.0)
    F = jnp.maximum(_dot(h, w1c[...]) + b1c[...], 0.0)
    E = jnp.maximum(_dot(xyz10, wx1[...]) + bx1[...], 0.0)
    A = jnp.maximum(_dot(E, w2a_e[...]) + _dot(F, w2a_f[...]) + b2a[...], 0.0)
    A = jnp.maximum(_dot(A, w2b[...]) + b2b[...], 0.0)
    o_ref[...] = _softmax_sum(A, F, bs)


def _phase2_body(g_ref, p_ref, f1_ref, wx2, bx2, w3a_e, w3a_f1, w3a_p, b3a,
                 w3b, b3b, o_ref, *, bs):
    pg, xyz10 = _pair_geom(g_ref, p_ref, bs)
    f1 = f1_ref[...]
    f1r = jnp.broadcast_to(f1[:, None, :], (bs, K, 64)).reshape(bs * K, 64)
    E = jnp.maximum(_dot(xyz10, wx2[...]) + bx2[...], 0.0)
    A = jnp.maximum(_dot(E, w3a_e[...]) + _dot(f1r, w3a_f1[...])
                    + _dot(pg, w3a_p[...]) + b3a[...], 0.0)
    A = jnp.maximum(_dot(A, w3b[...]) + b3b[...], 0.0)
    o_ref[...] = _softmax_sum(A, pg, bs)


def _full(shape_arr):
    return pl.BlockSpec(shape_arr.shape, lambda g: (0,) * shape_arr.ndim)


def _run_mlp(body, Garr, pts, f1, weights, bs):
    M = Garr.shape[0]
    rows = bs * K
    grid = (M // rows,)
    in_specs = [pl.BlockSpec((rows, C_TAB), lambda g: (g, 0)),
                pl.BlockSpec((bs, 3), lambda g: (g, 0)),
                pl.BlockSpec((bs, 64), lambda g: (g, 0))]
    in_specs += [_full(w) for w in weights]
    return pl.pallas_call(
        functools.partial(body, bs=bs),
        grid=grid,
        in_specs=in_specs,
        out_specs=pl.BlockSpec((bs, 64), lambda g: (g, 0)),
        out_shape=jax.ShapeDtypeStruct((M // K, 64), jnp.float32),
    )(Garr, pts, f1, *weights)


def kernel(xyz1, feature1, xyz2, feature2, mlp1_params, mlpxyz1_params,
           mlpxyz2_params, mlp2_params, mlp3_params):
    B, _, S = xyz1.shape
    N = xyz2.shape[2]
    SQ = B * S
    bs = 128
    bpb = S // QB

    xyz1_t = jnp.transpose(xyz1, (0, 2, 1))
    xyz2_t = jnp.transpose(xyz2, (0, 2, 1))
    f1_flat = jnp.transpose(feature1, (0, 2, 1)).reshape(SQ, 64)
    f2_t = jnp.transpose(feature2, (0, 2, 1))
    p_flat = xyz1_t.reshape(SQ, 3)
    q3 = jnp.transpose(p_flat.reshape(SQ // QB, QB, 3), (0, 2, 1))

    pad1 = jnp.zeros((B, N, C_TAB - 67), jnp.float32)
    table1 = jnp.concatenate([f2_t, xyz2_t, pad1], axis=-1).reshape(B * N, C_TAB)

    (W1a, b1a), (W1b, b1b), (W1c, b1c) = mlp1_params
    (Wx1, bx1), = mlpxyz1_params
    (Wx2, bx2), = mlpxyz2_params
    (W2a, b2a), (W2b, b2b) = mlp2_params
    (W3a, b3a), (W3b, b3b) = mlp3_params
    weights1 = [W1a.T[:10], W1a.T[10:74], W1a.T[74:], b1a[None, :],
                W1b.T, b1b[None, :], W1c.T, b1c[None, :],
                Wx1.T, bx1[None, :],
                W2a.T[:64], W2a.T[64:], b2a[None, :], W2b.T, b2b[None, :]]
    weights2 = [Wx2.T, bx2[None, :],
                W3a.T[:64], W3a.T[64:128], W3a.T[128:], b3a[None, :],
                W3b.T, b3b[None, :]]

    keys1, thr1 = _dist_thresh(q3, xyz2_t, bpb)
    G1 = _sc_extract_gather(keys1, thr1.reshape(SQ // QB, QB), table1, SQ, S)
    O1 = _run_mlp(_phase1_body, G1, p_flat, f1_flat, weights1, bs)

    keys2, thr2 = _dist_thresh(q3, xyz1_t, bpb)
    pad2 = jnp.zeros((B, S, C_TAB - 67), jnp.float32)
    table2 = jnp.concatenate([O1.reshape(B, S, 64), xyz1_t, pad2],
                             axis=-1).reshape(SQ, C_TAB)
    G2 = _sc_extract_gather(keys2, thr2.reshape(SQ // QB, QB), table2, SQ, S)
    O2 = _run_mlp(_phase2_body, G2, p_flat, f1_flat, weights2, bs)

    return jnp.transpose(O2.reshape(B, S, 64), (0, 2, 1))


# speedup vs baseline: 1.0971x; 1.0971x over previous
import functools

import jax
import jax.numpy as jnp
from jax import lax
from jax.experimental import pallas as pl
from jax.experimental.pallas import tpu as pltpu
from jax.experimental.pallas import tpu_sc as plsc

K = 32
QB = 256
C_TAB = 128
MIN32 = -2147483648


def _dist_body(q_ref, db_ref, keys_ref, thr_ref):
    q = q_ref[0]
    db = db_ref[0]
    s1 = jnp.sum(q * q, axis=0, keepdims=True)
    s2 = jnp.sum(db * db, axis=1, keepdims=True)
    mm = lax.dot_general(db.astype(jnp.bfloat16), q.astype(jnp.bfloat16),
                         (((1,), (0,)), ((), ())),
                         preferred_element_type=jnp.float32)
    d = (s1 + s2) - 2.0 * mm
    x = lax.bitcast_convert_type(d, jnp.int32)
    skey = x ^ (lax.shift_right_arithmetic(x, 31) & 0x7FFFFFFF)

    def body(i, prefix):
        bitval = lax.shift_left(jnp.int32(1), 31 - i)
        cand = prefix | bitval
        cand_s = cand ^ MIN32
        cnt = jnp.sum((skey < cand_s).astype(jnp.int32), axis=0, keepdims=True)
        return jnp.where(cnt >= K, prefix, cand)

    prefix = lax.fori_loop(0, 32, body, jnp.zeros((1, QB), jnp.int32))
    keys_ref[0] = skey
    thr_ref[0] = prefix ^ MIN32


def _dist_thresh(q3, db, blocks_per_batch):
    G = q3.shape[0]
    N = db.shape[1]
    keys, thr = pl.pallas_call(
        _dist_body,
        grid=(G,),
        in_specs=[pl.BlockSpec((1, 3, QB), lambda g: (g, 0, 0)),
                  pl.BlockSpec((1, N, 3),
                               lambda g, _n=blocks_per_batch: (g // _n, 0, 0))],
        out_specs=[pl.BlockSpec((1, N, QB), lambda g: (g, 0, 0)),
                   pl.BlockSpec((1, 1, QB), lambda g: (g, 0, 0))],
        out_shape=[jax.ShapeDtypeStruct((G, N, QB), jnp.int32),
                   jax.ShapeDtypeStruct((G, 1, QB), jnp.int32)],
    )(q3, db)
    return keys, thr


CCH = 64
SBQ = 4


def _sc_body(keys_hbm, thr_hbm, table_hbm, out_hbm,
             keys_v, thr_v, bufA, bufB, cntA_v, cntB_v, idx_v, rows_v,
             semg, semw, *, n_cand, q_total, s_batch):
    nc = 2
    wid = lax.axis_index("s") * nc + lax.axis_index("c")
    blk = wid
    base = blk * QB
    boff = (base // s_batch) * n_cand
    lane = lax.broadcasted_iota(jnp.int32, (16,), 0)
    zeros16 = jnp.zeros((16,), jnp.int32)

    pltpu.sync_copy(thr_hbm.at[pl.ds(blk, 1), :], thr_v)

    def zrow(i, _):
        cntA_v[i] = zeros16
        cntB_v[i] = zeros16
        return 0

    lax.fori_loop(0, 16, zrow, 0)

    def chunk_body(ch, _):
        pltpu.sync_copy(keys_hbm.at[pl.ds(blk, 1), pl.ds(ch * CCH, CCH), :],
                        keys_v)

        for qgg in range(4):
            qgs = [qgg * 4 + u for u in range(4)]
            ts = [thr_v[0, pl.ds(qg * 16, 16)] for qg in qgs]
            a49s = [(qg * 16 + lane) * 49 for qg in qgs]
            a65s = [(qg * 16 + lane) * 65 for qg in qgs]

            def cbody(c, carry, *, _ts=ts, _a49s=a49s, _a65s=a65s, _qgs=qgs):
                cAs = list(carry[:4])
                cBs = list(carry[4:])
                pos = lax.broadcast(c + ch * CCH + boff, (16,))
                for u in range(4):
                    k = keys_v[0, c, pl.ds(_qgs[u] * 16, 16)]
                    mlt = k < _ts[u]
                    meq = k == _ts[u]
                    plsc.store_scatter(bufA, [_a49s[u] + cAs[u]], pos,
                                       mask=mlt)
                    plsc.store_scatter(bufB, [_a65s[u] + cBs[u]], pos,
                                       mask=meq)
                    cAs[u] = cAs[u] + mlt.astype(jnp.int32)
                    cBs[u] = jnp.minimum(cBs[u] + meq.astype(jnp.int32), 64)
                return tuple(cAs) + tuple(cBs)

            carry0 = tuple(cntA_v[qg] for qg in qgs) + tuple(
                cntB_v[qg] for qg in qgs)
            res = plsc.parallel_loop(0, CCH, unroll=4, carry=carry0)(cbody)
            for u in range(4):
                cntA_v[qgs[u]] = res[u]
                cntB_v[qgs[u]] = res[4 + u]
        return 0

    lax.fori_loop(0, n_cand // CCH, chunk_body, 0)

    def mq(q, _):
        ca_vec = cntA_v[q // 16]
        ca = ca_vec[lax.broadcast(q % 16, (16,))]
        for half in range(2):
            j = lane + 16 * half
            vA = plsc.load_gather(bufA, [q * 49 + j])
            jb = jnp.maximum(j - ca, 0)
            vB = plsc.load_gather(bufB, [q * 65 + jb])
            sel = jnp.where(j < ca, vA, vB)
            idx_v[q, pl.ds(16 * half, 16)] = sel
        return 0

    lax.fori_loop(0, QB, mq, 0)

    nsb = QB // SBQ
    gbytes_dst = lambda s, r: rows_v.at[s, pl.ds(r * K, K)]

    def fire_g(i, s):
        for r in range(SBQ):
            pltpu.async_copy(table_hbm.at[idx_v.at[i * SBQ + r]],
                             gbytes_dst(s, r), semg[s])

    def drain_g(s):
        for r in range(SBQ):
            pltpu.make_async_copy(table_hbm.at[pl.ds(0, K)],
                                  gbytes_dst(s, r), semg[s]).wait()

    def fire_w(i, s):
        pltpu.async_copy(rows_v.at[s],
                         out_hbm.at[pl.ds((base + i * SBQ) * K, SBQ * K)],
                         semw[s])

    def drain_w(s):
        pltpu.make_async_copy(rows_v.at[s],
                              out_hbm.at[pl.ds(base * K, SBQ * K)],
                              semw[s]).wait()

    fire_g(0, 0)

    def pipe(g, _):
        @pl.when(g > 0)
        def _():
            drain_w(1)
        fire_g(2 * g + 1, 1)
        drain_g(0)
        fire_w(2 * g, 0)

        @pl.when(g < nsb // 2 - 1)
        def _():
            drain_w(0)
            fire_g(2 * g + 2, 0)

        drain_g(1)
        fire_w(2 * g + 1, 1)
        return 0

    lax.fori_loop(0, nsb // 2, pipe, 0)
    drain_w(0)
    drain_w(1)


def _sc_extract_gather(keys, thr, table, q_total, s_batch):
    n_cand = keys.shape[1]
    body = functools.partial(_sc_body, n_cand=n_cand, q_total=q_total,
                             s_batch=s_batch)
    fn = functools.partial(
        pl.kernel, body,
        mesh=plsc.VectorSubcoreMesh(core_axis_name="c", subcore_axis_name="s"),
        compiler_params=pltpu.CompilerParams(needs_layout_passes=False),
        out_type=jax.ShapeDtypeStruct((q_total * K, C_TAB), jnp.float32),
        scratch_types=[
            pltpu.VMEM((1, CCH, QB), jnp.int32),
            pltpu.VMEM((1, QB), jnp.int32),
            pltpu.VMEM((QB * 49,), jnp.int32),
            pltpu.VMEM((QB * 65,), jnp.int32),
            pltpu.VMEM((16, 16), jnp.int32),
            pltpu.VMEM((16, 16), jnp.int32),
            pltpu.VMEM((QB, K), jnp.int32),
            pltpu.VMEM((2, SBQ * K, C_TAB), jnp.float32),
            [pltpu.SemaphoreType.DMA, pltpu.SemaphoreType.DMA],
            [pltpu.SemaphoreType.DMA, pltpu.SemaphoreType.DMA],
        ])()
    return fn(keys, thr, table)


def _dot(a, b):
    return lax.dot_general(a.astype(jnp.bfloat16), b.astype(jnp.bfloat16),
                           (((1,), (0,)), ((), ())),
                           preferred_element_type=jnp.float32)


def _pair_geom(g_ref, p_ref, bs):
    G = g_ref[...]
    p = p_ref[...]
    p3 = jnp.broadcast_to(p[:, None, :], (bs, K, 3)).reshape(bs * K, 3)
    qx = G[:, 64:67]
    diff = qx - p3
    euc = jnp.sqrt(jnp.sum(diff * diff, axis=1, keepdims=True) + 1e-20)
    xyz10 = jnp.concatenate([p3, qx, diff, euc], axis=1)
    return G[:, 0:64], xyz10


def _softmax_sum(A, F, bs):
    A3 = A.reshape(bs, K, 64)
    F3 = F.reshape(bs, K, 64)
    m = jnp.max(A3, axis=1, keepdims=True)
    e = jnp.exp(A3 - m)
    w = e / jnp.sum(e, axis=1, keepdims=True)
    return jnp.sum(w * F3, axis=1)


def _phase1_body(g_ref, p_ref, f1_ref, w1a_x, w1a_f1, w1a_f2, b1a, w1b, b1b,
                 w1c, b1c, wx1, bx1, w2a_e, w2a_f, b2a, w2b, b2b, o_ref, *, bs):
    qf, xyz10 = _pair_geom(g_ref, p_ref, bs)
    f1 = f1_ref[...]
    f1r = jnp.broadcast_to(f1[:, None, :], (bs, K, 64)).reshape(bs * K, 64)
    h = jnp.maximum(_dot(xyz10, w1a_x[...]) + _dot(f1r, w1a_f1[...])
                    + _dot(qf, w1a_f2[...]) + b1a[...], 0.0)
    h = jnp.maximum(_dot(h, w1b[...]) + b1b[...], 0.0)
    F = jnp.maximum(_dot(h, w1c[...]) + b1c[...], 0.0)
    E = jnp.maximum(_dot(xyz10, wx1[...]) + bx1[...], 0.0)
    A = jnp.maximum(_dot(E, w2a_e[...]) + _dot(F, w2a_f[...]) + b2a[...], 0.0)
    A = jnp.maximum(_dot(A, w2b[...]) + b2b[...], 0.0)
    o_ref[...] = _softmax_sum(A, F, bs)


def _phase2_body(g_ref, p_ref, f1_ref, wx2, bx2, w3a_e, w3a_f1, w3a_p, b3a,
                 w3b, b3b, o_ref, *, bs):
    pg, xyz10 = _pair_geom(g_ref, p_ref, bs)
    f1 = f1_ref[...]
    f1r = jnp.broadcast_to(f1[:, None, :], (bs, K, 64)).reshape(bs * K, 64)
    E = jnp.maximum(_dot(xyz10, wx2[...]) + bx2[...], 0.0)
    A = jnp.maximum(_dot(E, w3a_e[...]) + _dot(f1r, w3a_f1[...])
                    + _dot(pg, w3a_p[...]) + b3a[...], 0.0)
    A = jnp.maximum(_dot(A, w3b[...]) + b3b[...], 0.0)
    o_ref[...] = _softmax_sum(A, pg, bs)


def _full(shape_arr):
    return pl.BlockSpec(shape_arr.shape, lambda g: (0,) * shape_arr.ndim)


def _run_mlp(body, Garr, pts, f1, weights, bs):
    M = Garr.shape[0]
    rows = bs * K
    grid = (M // rows,)
    in_specs = [pl.BlockSpec((rows, C_TAB), lambda g: (g, 0)),
                pl.BlockSpec((bs, 3), lambda g: (g, 0)),
                pl.BlockSpec((bs, 64), lambda g: (g, 0))]
    in_specs += [_full(w) for w in weights]
    return pl.pallas_call(
        functools.partial(body, bs=bs),
        grid=grid,
        in_specs=in_specs,
        out_specs=pl.BlockSpec((bs, 64), lambda g: (g, 0)),
        out_shape=jax.ShapeDtypeStruct((M // K, 64), jnp.float32),
    )(Garr, pts, f1, *weights)


def kernel(xyz1, feature1, xyz2, feature2, mlp1_params, mlpxyz1_params,
           mlpxyz2_params, mlp2_params, mlp3_params):
    B, _, S = xyz1.shape
    N = xyz2.shape[2]
    SQ = B * S
    bs = 128
    bpb = S // QB

    xyz1_t = jnp.transpose(xyz1, (0, 2, 1))
    xyz2_t = jnp.transpose(xyz2, (0, 2, 1))
    f1_flat = jnp.transpose(feature1, (0, 2, 1)).reshape(SQ, 64)
    f2_t = jnp.transpose(feature2, (0, 2, 1))
    p_flat = xyz1_t.reshape(SQ, 3)
    q3 = jnp.transpose(p_flat.reshape(SQ // QB, QB, 3), (0, 2, 1))

    pad1 = jnp.zeros((B, N, C_TAB - 67), jnp.float32)
    table1 = jnp.concatenate([f2_t, xyz2_t, pad1], axis=-1).reshape(B * N, C_TAB)

    (W1a, b1a), (W1b, b1b), (W1c, b1c) = mlp1_params
    (Wx1, bx1), = mlpxyz1_params
    (Wx2, bx2), = mlpxyz2_params
    (W2a, b2a), (W2b, b2b) = mlp2_params
    (W3a, b3a), (W3b, b3b) = mlp3_params
    weights1 = [W1a.T[:10], W1a.T[10:74], W1a.T[74:], b1a[None, :],
                W1b.T, b1b[None, :], W1c.T, b1c[None, :],
                Wx1.T, bx1[None, :],
                W2a.T[:64], W2a.T[64:], b2a[None, :], W2b.T, b2b[None, :]]
    weights2 = [Wx2.T, bx2[None, :],
                W3a.T[:64], W3a.T[64:128], W3a.T[128:], b3a[None, :],
                W3b.T, b3b[None, :]]

    keys1, thr1 = _dist_thresh(q3, xyz2_t, bpb)
    G1 = _sc_extract_gather(keys1, thr1.reshape(SQ // QB, QB), table1, SQ, S)
    O1 = _run_mlp(_phase1_body, G1, p_flat, f1_flat, weights1, bs)

    keys2, thr2 = _dist_thresh(q3, xyz1_t, bpb)
    pad2 = jnp.zeros((B, S, C_TAB - 67), jnp.float32)
    table2 = jnp.concatenate([O1.reshape(B, S, 64), xyz1_t, pad2],
                             axis=-1).reshape(SQ, C_TAB)
    G2 = _sc_extract_gather(keys2, thr2.reshape(SQ // QB, QB), table2, SQ, S)
    O2 = _run_mlp(_phase2_body, G2, p_flat, f1_flat, weights2, bs)

    return jnp.transpose(O2.reshape(B, S, 64), (0, 2, 1))
